# baseline (speedup 1.0000x reference)
import functools

import jax
import jax.numpy as jnp
from jax import lax
from jax.experimental import pallas as pl
from jax.experimental.pallas import tpu as pltpu
from jax.experimental.pallas import tpu_sc as plsc

N = 1000000
D = 64
DP = 128
B = 16384
K = 10

NC = 2
NS = 16
NW = NC * NS
EPW = B // NW
C = 64
NCHUNK = EPW // C
LANES = 16
GROUPS = C // LANES

TBLK = 8192


def _tc_transpose(tt):
  def body(in_ref, o_ref):
    xt = in_ref[...].T
    o_ref[...] = jnp.concatenate([xt, jnp.zeros_like(xt)], axis=1)

  nblk = (N + TBLK - 1) // TBLK
  return pl.pallas_call(
      body,
      grid=(nblk,),
      in_specs=[pl.BlockSpec((D, TBLK), lambda i: (0, i))],
      out_specs=pl.BlockSpec((TBLK, DP), lambda i: (i, 0)),
      out_shape=jax.ShapeDtypeStruct((N, DP), jnp.float32),
  )(tt)


def _sc_scores(target_p, context_p, src, dst, negs_flat):
  mesh = plsc.VectorSubcoreMesh(core_axis_name="c", subcore_axis_name="s")

  @functools.partial(
      pl.kernel,
      out_type=(
          jax.ShapeDtypeStruct((B,), jnp.float32),
          jax.ShapeDtypeStruct((B * K,), jnp.float32),
      ),
      mesh=mesh,
      scratch_types=[
          pltpu.VMEM((C,), jnp.int32),
          pltpu.VMEM((C,), jnp.int32),
          pltpu.VMEM((K * C,), jnp.int32),
          pltpu.VMEM((C, DP), jnp.float32),
          pltpu.VMEM((C, DP), jnp.float32),
          pltpu.VMEM((K * C, DP), jnp.float32),
          pltpu.VMEM((C,), jnp.float32),
          pltpu.VMEM((K * C,), jnp.float32),
          pltpu.SemaphoreType.DMA,
      ],
      compiler_params=pltpu.CompilerParams(needs_layout_passes=False),
  )
  def k(target_hbm, context_hbm, src_hbm, dst_hbm, negs_hbm,
        pos_out, neg_out, idx_src, idx_dst, idx_neg,
        src_rows, dst_rows, neg_rows, pos_buf, neg_buf, sem):
    wid = lax.axis_index("s") * NC + lax.axis_index("c")
    iota16 = lax.iota(jnp.int32, LANES)

    for c in range(NCHUNK):
      base = wid * EPW + c * C
      pltpu.sync_copy(src_hbm.at[pl.ds(base, C)], idx_src)
      pltpu.sync_copy(dst_hbm.at[pl.ds(base, C)], idx_dst)
      pltpu.sync_copy(negs_hbm.at[pl.ds(base * K, K * C)], idx_neg)

      copies = [
          pltpu.async_copy(target_hbm.at[idx_src], src_rows, sem),
          pltpu.async_copy(context_hbm.at[idx_dst], dst_rows, sem),
      ]
      for j in range(K * C // 128):
        copies.append(
            pltpu.async_copy(context_hbm.at[idx_neg.at[pl.ds(j * 128, 128)]],
                             neg_rows.at[pl.ds(j * 128, 128)], sem))
      for h in copies:
        h.wait()

      def group_body(g, carry):
        sl = pl.ds(g * LANES, LANES)
        row16 = g * LANES + iota16
        nrows = [row16 * K + kk for kk in range(K)]
        zero = jnp.zeros((LANES,), jnp.float32)

        def d_body(d, accs):
          col = jnp.broadcast_to(d, (LANES,))
          sv = plsc.load_gather(src_rows, [row16, col])
          dv = plsc.load_gather(dst_rows, [row16, col])
          new = [accs[0] + sv * dv]
          for kk in range(K):
            nv = plsc.load_gather(neg_rows, [nrows[kk], col])
            new.append(accs[kk + 1] + sv * nv)
          return tuple(new)

        accs = lax.fori_loop(0, D, d_body, (zero,) * (K + 1))
        pos_buf[sl] = accs[0]
        for kk in range(K):
          neg_buf[pl.ds(kk * C + g * LANES, LANES)] = accs[kk + 1]
        return carry

      lax.fori_loop(0, GROUPS, group_body, None)

      pltpu.sync_copy(pos_buf, pos_out.at[pl.ds(base, C)])
      pltpu.sync_copy(neg_buf, neg_out.at[pl.ds(base * K, K * C)])

  return k(target_p, context_p, src, dst, negs_flat)


def _tc_loss(pos, neg):
  def body(pos_ref, neg_ref, o_ref):
    p = pos_ref[...]
    n = neg_ref[...]
    pos_loss = -jnp.mean(jnp.log(jax.nn.sigmoid(p) + 1e-15))
    neg_loss = -jnp.mean(jnp.log(jax.nn.sigmoid(-n) + 1e-15))
    o_ref[0, 0] = pos_loss + neg_loss

  out = pl.pallas_call(
      body,
      out_shape=jax.ShapeDtypeStruct((1, 1), jnp.float32),
      out_specs=pl.BlockSpec(memory_space=pltpu.SMEM),
  )(pos, neg)
  return out[0, 0]


@jax.jit
def kernel(src, dst, negs, target, context):
  src = src.astype(jnp.int32)
  dst = dst.astype(jnp.int32)
  negs_flat = negs.astype(jnp.int32).reshape(B * K)
  target_p = _tc_transpose(target.T)
  context_p = _tc_transpose(context.T)
  pos, neg = _sc_scores(target_p, context_p, src, dst, negs_flat)
  return _tc_loss(pos.reshape(B // 128, 128), neg.reshape(B * K // 128, 128))


# speedup vs baseline: 1.4128x; 1.4128x over previous
import functools

import jax
import jax.numpy as jnp
from jax import lax
from jax.experimental import pallas as pl
from jax.experimental.pallas import tpu as pltpu
from jax.experimental.pallas import tpu_sc as plsc

N = 1000000
D = 64
DP = 128
B = 16384
K = 10

NC = 2
NS = 16
NW = NC * NS
EPW = B // NW
C = 128
NCHUNK = EPW // C
LANES = 16
GROUPS = C // LANES

TBLK = 8192
HBLK = TBLK // 2
NBLK = (N + TBLK - 1) // TBLK
NROW = NBLK * TBLK


def _tc_transpose(tt):
  def body(in_ref, o_ref):
    x = in_ref[...]
    o_ref[...] = jnp.concatenate(
        [x[:, :HBLK].T, x[:, HBLK:].T], axis=1)

  return pl.pallas_call(
      body,
      grid=(NBLK,),
      in_specs=[pl.BlockSpec((D, TBLK), lambda i: (0, i))],
      out_specs=pl.BlockSpec((HBLK, DP), lambda i: (i, 0)),
      out_shape=jax.ShapeDtypeStruct((NROW // 2, DP), jnp.float32),
  )(tt)


def _sc_scores(target_p, context_p, src, dst, negs_flat):
  mesh = plsc.VectorSubcoreMesh(core_axis_name="c", subcore_axis_name="s")

  @functools.partial(
      pl.kernel,
      out_type=(
          jax.ShapeDtypeStruct((B,), jnp.float32),
          jax.ShapeDtypeStruct((B * K,), jnp.float32),
      ),
      mesh=mesh,
      scratch_types=[
          pltpu.VMEM((C,), jnp.int32),
          pltpu.VMEM((C,), jnp.int32),
          pltpu.VMEM((K * C,), jnp.int32),
          pltpu.VMEM((C, D), jnp.float32),
          pltpu.VMEM((C, D), jnp.float32),
          pltpu.VMEM((K * C, D), jnp.float32),
          pltpu.VMEM((C,), jnp.float32),
          pltpu.VMEM((K * C,), jnp.float32),
          pltpu.SemaphoreType.DMA,
      ],
      compiler_params=pltpu.CompilerParams(
          needs_layout_passes=False, use_tc_tiling_on_sc=False),
  )
  def k(target_hbm, context_hbm, src_hbm, dst_hbm, negs_hbm,
        pos_out, neg_out, idx_src, idx_dst, idx_neg,
        src_rows, dst_rows, neg_rows, pos_buf, neg_buf, sem):
    wid = lax.axis_index("s") * NC + lax.axis_index("c")
    iota16 = lax.iota(jnp.int32, LANES)

    def to_row(n):
      return (n & ~(TBLK - 1)) + ((n & (HBLK - 1)) << 1) + ((n >> 12) & 1)

    for c in range(NCHUNK):
      base = wid * EPW + c * C
      pltpu.sync_copy(src_hbm.at[pl.ds(base, C)], idx_src)
      pltpu.sync_copy(dst_hbm.at[pl.ds(base, C)], idx_dst)
      pltpu.sync_copy(negs_hbm.at[pl.ds(base * K, K * C)], idx_neg)

      def perm_body(i, carry):
        sl = pl.ds(i * LANES, LANES)
        idx_src[sl] = to_row(idx_src[sl])
        idx_dst[sl] = to_row(idx_dst[sl])
        return carry

      lax.fori_loop(0, C // LANES, perm_body, None)

      def nperm_body(i, carry):
        sl = pl.ds(i * LANES, LANES)
        idx_neg[sl] = to_row(idx_neg[sl])
        return carry

      lax.fori_loop(0, K * C // LANES, nperm_body, None)

      copies = [
          pltpu.async_copy(target_hbm.at[idx_src], src_rows, sem),
          pltpu.async_copy(context_hbm.at[idx_dst], dst_rows, sem),
      ]
      for j in range(K * C // 128):
        copies.append(
            pltpu.async_copy(context_hbm.at[idx_neg.at[pl.ds(j * 128, 128)]],
                             neg_rows.at[pl.ds(j * 128, 128)], sem))
      for h in copies:
        h.wait()

      def group_body(g, carry):
        def edge_body(i, accs):
          e = g * LANES + i
          lanemask = iota16 == i
          s = [src_rows[e, pl.ds(j * LANES, LANES)] for j in range(D // LANES)]
          dv = [dst_rows[e, pl.ds(j * LANES, LANES)] for j in range(D // LANES)]
          p = s[0] * dv[0]
          for j in range(1, D // LANES):
            p = p + s[j] * dv[j]
          new = [jnp.where(lanemask, jnp.sum(p), accs[0])]
          for kk in range(K):
            r = e * K + kk
            q = s[0] * neg_rows[r, pl.ds(0, LANES)]
            for j in range(1, D // LANES):
              q = q + s[j] * neg_rows[r, pl.ds(j * LANES, LANES)]
            new.append(jnp.where(lanemask, jnp.sum(q), accs[kk + 1]))
          return tuple(new)

        zero = jnp.zeros((LANES,), jnp.float32)
        accs = lax.fori_loop(0, LANES, edge_body, (zero,) * (K + 1))
        pos_buf[pl.ds(g * LANES, LANES)] = accs[0]
        for kk in range(K):
          neg_buf[pl.ds(kk * C + g * LANES, LANES)] = accs[kk + 1]
        return carry

      lax.fori_loop(0, GROUPS, group_body, None)

      pltpu.sync_copy(pos_buf, pos_out.at[pl.ds(base, C)])
      pltpu.sync_copy(neg_buf, neg_out.at[pl.ds(base * K, K * C)])

  return k(target_p, context_p, src, dst, negs_flat)


def _tc_loss(pos, neg):
  def body(pos_ref, neg_ref, o_ref):
    p = pos_ref[...]
    n = neg_ref[...]
    pos_loss = -jnp.mean(jnp.log(jax.nn.sigmoid(p) + 1e-15))
    neg_loss = -jnp.mean(jnp.log(jax.nn.sigmoid(-n) + 1e-15))
    o_ref[0, 0] = pos_loss + neg_loss

  out = pl.pallas_call(
      body,
      out_shape=jax.ShapeDtypeStruct((1, 1), jnp.float32),
      out_specs=pl.BlockSpec(memory_space=pltpu.SMEM),
  )(pos, neg)
  return out[0, 0]


@jax.jit
def kernel(src, dst, negs, target, context):
  src = src.astype(jnp.int32)
  dst = dst.astype(jnp.int32)
  negs_flat = negs.astype(jnp.int32).reshape(B * K)
  target_p = _tc_transpose(target.T).reshape(NROW, D)
  context_p = _tc_transpose(context.T).reshape(NROW, D)
  pos, neg = _sc_scores(target_p, context_p, src, dst, negs_flat)
  return _tc_loss(pos.reshape(B // 128, 128), neg.reshape(B * K // 128, 128))


# speedup vs baseline: 1.8085x; 1.2800x over previous
import functools

import jax
import jax.numpy as jnp
from jax import lax
from jax.experimental import pallas as pl
from jax.experimental.pallas import tpu as pltpu
from jax.experimental.pallas import tpu_sc as plsc

N = 1000000
D = 64
DP = 128
B = 16384
K = 10

NC = 2
NS = 16
NW = NC * NS
EPW = B // NW
C = 128
NCHUNK = EPW // C
LANES = 16
GROUPS = C // LANES

TBLK = 8192
HBLK = TBLK // 2
NBLK = (N + TBLK - 1) // TBLK
NROW = NBLK * TBLK


def _tc_transpose(tt):
  def body(in_ref, o_ref):
    x = in_ref[...]
    x2 = jnp.concatenate([x[:, :HBLK], x[:, HBLK:]], axis=0)
    r = lax.broadcasted_iota(jnp.int32, (DP, DP), 0)
    cc = lax.broadcasted_iota(jnp.int32, (DP, DP), 1)
    eye = jnp.where(r == cc, 1.0, 0.0).astype(jnp.float32)
    o_ref[...] = lax.dot_general(x2, eye, (((0,), (0,)), ((), ())),
                                 preferred_element_type=jnp.float32)

  return pl.pallas_call(
      body,
      grid=(NBLK,),
      in_specs=[pl.BlockSpec((D, TBLK), lambda i: (0, i))],
      out_specs=pl.BlockSpec((HBLK, DP), lambda i: (i, 0)),
      out_shape=jax.ShapeDtypeStruct((NROW // 2, DP), jnp.float32),
      compiler_params=pltpu.CompilerParams(fuse_transposed_lhs_in_matmul=True),
  )(tt)


def _sc_scores(target_p, context_p, src, dst, negs_flat):
  mesh = plsc.VectorSubcoreMesh(core_axis_name="c", subcore_axis_name="s")

  @functools.partial(
      pl.kernel,
      out_type=(
          jax.ShapeDtypeStruct((B,), jnp.float32),
          jax.ShapeDtypeStruct((B * K,), jnp.float32),
      ),
      mesh=mesh,
      scratch_types=[
          pltpu.VMEM((C,), jnp.int32),
          pltpu.VMEM((C,), jnp.int32),
          pltpu.VMEM((K * C,), jnp.int32),
          pltpu.VMEM((C, D), jnp.float32),
          pltpu.VMEM((C, D), jnp.float32),
          pltpu.VMEM((K * C, D), jnp.float32),
          pltpu.VMEM((C,), jnp.float32),
          pltpu.VMEM((K * C,), jnp.float32),
          pltpu.SemaphoreType.DMA,
      ],
      compiler_params=pltpu.CompilerParams(
          needs_layout_passes=False, use_tc_tiling_on_sc=False),
  )
  def k(target_hbm, context_hbm, src_hbm, dst_hbm, negs_hbm,
        pos_out, neg_out, idx_src, idx_dst, idx_neg,
        src_rows, dst_rows, neg_rows, pos_buf, neg_buf, sem):
    wid = lax.axis_index("s") * NC + lax.axis_index("c")
    iota16 = lax.iota(jnp.int32, LANES)

    def to_row(n):
      return (n & ~(TBLK - 1)) + ((n & (HBLK - 1)) << 1) + ((n >> 12) & 1)

    for c in range(NCHUNK):
      base = wid * EPW + c * C
      pltpu.sync_copy(src_hbm.at[pl.ds(base, C)], idx_src)
      pltpu.sync_copy(dst_hbm.at[pl.ds(base, C)], idx_dst)
      pltpu.sync_copy(negs_hbm.at[pl.ds(base * K, K * C)], idx_neg)

      def perm_body(i, carry):
        sl = pl.ds(i * LANES, LANES)
        idx_src[sl] = to_row(idx_src[sl])
        idx_dst[sl] = to_row(idx_dst[sl])
        return carry

      lax.fori_loop(0, C // LANES, perm_body, None)

      def nperm_body(i, carry):
        sl = pl.ds(i * LANES, LANES)
        idx_neg[sl] = to_row(idx_neg[sl])
        return carry

      lax.fori_loop(0, K * C // LANES, nperm_body, None)

      copies = [
          pltpu.async_copy(target_hbm.at[idx_src], src_rows, sem),
          pltpu.async_copy(context_hbm.at[idx_dst], dst_rows, sem),
      ]
      for j in range(K * C // 128):
        copies.append(
            pltpu.async_copy(context_hbm.at[idx_neg.at[pl.ds(j * 128, 128)]],
                             neg_rows.at[pl.ds(j * 128, 128)], sem))
      for h in copies:
        h.wait()

      def group_body(g, carry):
        def edge_body(i, accs):
          e = g * LANES + i
          lanemask = iota16 == i
          s = [src_rows[e, pl.ds(j * LANES, LANES)] for j in range(D // LANES)]
          dv = [dst_rows[e, pl.ds(j * LANES, LANES)] for j in range(D // LANES)]
          p = s[0] * dv[0]
          for j in range(1, D // LANES):
            p = p + s[j] * dv[j]
          new = [jnp.where(lanemask, jnp.sum(p), accs[0])]
          for kk in range(K):
            r = e * K + kk
            q = s[0] * neg_rows[r, pl.ds(0, LANES)]
            for j in range(1, D // LANES):
              q = q + s[j] * neg_rows[r, pl.ds(j * LANES, LANES)]
            new.append(jnp.where(lanemask, jnp.sum(q), accs[kk + 1]))
          return tuple(new)

        zero = jnp.zeros((LANES,), jnp.float32)
        accs = lax.fori_loop(0, LANES, edge_body, (zero,) * (K + 1))
        pos_buf[pl.ds(g * LANES, LANES)] = accs[0]
        for kk in range(K):
          neg_buf[pl.ds(kk * C + g * LANES, LANES)] = accs[kk + 1]
        return carry

      lax.fori_loop(0, GROUPS, group_body, None)

      pltpu.sync_copy(pos_buf, pos_out.at[pl.ds(base, C)])
      pltpu.sync_copy(neg_buf, neg_out.at[pl.ds(base * K, K * C)])

  return k(target_p, context_p, src, dst, negs_flat)


def _tc_loss(pos, neg):
  def body(pos_ref, neg_ref, o_ref):
    p = pos_ref[...]
    n = neg_ref[...]
    pos_loss = -jnp.mean(jnp.log(jax.nn.sigmoid(p) + 1e-15))
    neg_loss = -jnp.mean(jnp.log(jax.nn.sigmoid(-n) + 1e-15))
    o_ref[0, 0] = pos_loss + neg_loss

  out = pl.pallas_call(
      body,
      out_shape=jax.ShapeDtypeStruct((1, 1), jnp.float32),
      out_specs=pl.BlockSpec(memory_space=pltpu.SMEM),
  )(pos, neg)
  return out[0, 0]


@jax.jit
def kernel(src, dst, negs, target, context):
  src = src.astype(jnp.int32)
  dst = dst.astype(jnp.int32)
  negs_flat = negs.astype(jnp.int32).reshape(B * K)
  target_p = _tc_transpose(target.T).reshape(NROW, D)
  context_p = _tc_transpose(context.T).reshape(NROW, D)
  pos, neg = _sc_scores(target_p, context_p, src, dst, negs_flat)
  return _tc_loss(pos.reshape(B // 128, 128), neg.reshape(B * K // 128, 128))
